# baseline (speedup 1.0000x reference)
import functools

import jax
import jax.numpy as jnp
from jax import lax
from jax.experimental import pallas as pl
from jax.experimental.pallas import tpu as pltpu
from jax.experimental.pallas import tpu_sc as plsc

ROW = 1000
GROUP = 16
NW = 32
INT_MIN = jnp.int32(-2147483648)
LOW10 = jnp.int32(1023)
NLOW10 = jnp.int32(-1024)
DIGITS = [(0, 1023), (10, 255), (18, 127), (25, 127)]
HB_OFF = [0, 0, 4096, 6144]


def _make(n_rows):
    rows_per_w = n_rows // NW
    groups = rows_per_w // GROUP
    stag_words = GROUP * ROW
    mesh = plsc.VectorSubcoreMesh(core_axis_name="c", subcore_axis_name="s")

    @functools.partial(
        pl.kernel,
        out_type=jax.ShapeDtypeStruct((n_rows * ROW,), jnp.int32),
        mesh=mesh,
        scratch_types=[
            pltpu.VMEM((stag_words,), jnp.int32),
            pltpu.VMEM((stag_words,), jnp.int32),
            pltpu.VMEM((stag_words,), jnp.int32),
            pltpu.VMEM((stag_words,), jnp.int32),
            pltpu.VMEM((1024 * GROUP,), jnp.int32),
            pltpu.VMEM((512 * GROUP,), jnp.int32),
            pltpu.SemaphoreType.DMA,
            pltpu.SemaphoreType.DMA,
        ],
        compiler_params=pltpu.CompilerParams(needs_layout_passes=False),
    )
    def k(x_hbm, out_hbm, stag_in, stag_out, b1, b2, hista, histb,
          sem_in, sem_out):
        c = lax.axis_index("c")
        s = lax.axis_index("s")
        wid = s * 2 + c
        iota = lax.iota(jnp.int32, 16)
        zeros = jnp.zeros((16,), jnp.int32)
        ones = jnp.ones((16,), jnp.int32)

        def transform(bits):
            sgn = lax.shift_right_arithmetic(bits, 31)
            m = bits ^ (sgn | INT_MIN)
            return ~m

        def zero_loop(ref, nvregs):
            def body(b, _):
                for m in range(4):
                    ref[pl.ds((b * 4 + m) * 16, 16)] = zeros
                return 0

            lax.fori_loop(0, nvregs // 4, body, 0)

        def scan(ref, off, nbins):
            def body(blk, run):
                for m in range(4):
                    b = off + (blk * 4 + m) * 16
                    v = ref[pl.ds(b, 16)]
                    ref[pl.ds(b, 16)] = run
                    run = run + v
                return run

            lax.fori_loop(0, nbins // 4, body, zeros)

        zero_loop(hista, 1024)
        zero_loop(histb, 512)

        wbase = wid * rows_per_w * ROW
        pltpu.sync_copy(x_hbm.at[pl.ds(wbase, stag_words)], stag_in)

        def do_group(g, _):
            base = wbase + g * stag_words
            nbase = base + stag_words

            def ha_body(t, _):
                r = t & 15
                j = lax.shift_right_logical(t, 4)
                posv = j * 16 + iota
                src = jnp.minimum(r * ROW + posv, stag_words - 1)
                key = transform(plsc.load_gather(stag_in, [src]))
                msk = posv < ROW
                hidx = ((key & LOW10) << 4) + r
                plsc.addupdate_scatter(hista, [hidx], ones, mask=msk)
                return 0

            lax.fori_loop(0, GROUP * 63, ha_body, 0)
            scan(hista, 0, 1024)

            def radix_pass(src, dst, cur, hoff, p, kind):
                shift, mask = DIGITS[p]

                def body(t, _):
                    r = t & 15
                    j = lax.shift_right_logical(t, 4)
                    posv = j * 16 + iota
                    msk = posv < ROW
                    srcidx = jnp.minimum(r * ROW + posv, stag_words - 1)
                    key = plsc.load_gather(src, [srcidx])
                    if kind == 0:
                        key = transform(key)
                    d = lax.shift_right_logical(key, shift) & mask
                    hidx = (d << 4) + (r + hoff)
                    cnt, _ = plsc.scan_count(d, mask=msk)
                    bsv = plsc.load_gather(cur, [hidx])
                    pos = bsv + cnt.astype(jnp.int32) - 1
                    dv = jnp.minimum(r * ROW + pos, stag_words - 1)
                    if kind == 0:
                        plsc.store_scatter(
                            dst, [dv], (key & NLOW10) | posv, mask=msk)
                        for pp in (1, 2, 3):
                            sh, mk = DIGITS[pp]
                            dp = lax.shift_right_logical(key, sh) & mk
                            plsc.addupdate_scatter(
                                histb, [(dp << 4) + (r + HB_OFF[pp])],
                                ones, mask=msk)
                    elif kind == 1:
                        plsc.store_scatter(dst, [dv], key, mask=msk)
                    else:
                        plsc.store_scatter(dst, [dv], key & LOW10, mask=msk)
                    plsc.addupdate_scatter(cur, [hidx], ones, mask=msk)
                    return 0

                lax.fori_loop(0, GROUP * 63, body, 0)

            radix_pass(stag_in, b1, hista, 0, 0, 0)

            @pl.when(g + 1 < groups)
            def _():
                pltpu.make_async_copy(
                    x_hbm.at[pl.ds(nbase, stag_words)], stag_in,
                    sem_in).start()

            scan(histb, 0, 256)
            radix_pass(b1, b2, histb, HB_OFF[1], 1, 1)
            scan(histb, 4096, 128)
            radix_pass(b2, b1, histb, HB_OFF[2], 2, 1)
            scan(histb, 6144, 128)

            @pl.when(g > 0)
            def _():
                pltpu.make_async_copy(
                    stag_out, out_hbm.at[pl.ds(base - stag_words, stag_words)],
                    sem_out).wait()

            radix_pass(b1, stag_out, histb, HB_OFF[3], 3, 2)

            zero_loop(hista, 1024)
            zero_loop(histb, 512)

            pltpu.make_async_copy(
                stag_out, out_hbm.at[pl.ds(base, stag_words)],
                sem_out).start()

            @pl.when(g + 1 < groups)
            def _():
                pltpu.make_async_copy(
                    x_hbm.at[pl.ds(nbase, stag_words)], stag_in,
                    sem_in).wait()

            return 0

        lax.fori_loop(0, groups, do_group, 0)
        pltpu.make_async_copy(
            stag_out,
            out_hbm.at[pl.ds(wbase + (groups - 1) * stag_words, stag_words)],
            sem_out).wait()

    return k


@jax.jit
def kernel(inputs):
    n_rows = inputs.size // ROW
    xi = jax.lax.bitcast_convert_type(inputs, jnp.int32)
    out = _make(n_rows)(xi)
    return out.reshape(n_rows, ROW)


# speedup vs baseline: 2.8327x; 2.8327x over previous
import functools

import jax
import jax.numpy as jnp
from jax import lax
from jax.experimental import pallas as pl
from jax.experimental.pallas import tpu as pltpu
from jax.experimental.pallas import tpu_sc as plsc

ROW = 1000
PPAD = 1008
GROUP = 16
NW = 32
T17 = PPAD * 17
T16 = PPAD * 16
INT_MIN = jnp.int32(-2147483648)
LOW10 = jnp.int32(1023)
NLOW10 = jnp.int32(-1024)
DIGITS = [(0, 1023), (10, 255), (18, 127), (25, 127)]
HB_OFF = [0, 0, 4096, 6144]


def _make(n_rows):
    rows_per_w = n_rows // NW
    groups = rows_per_w // GROUP
    stag_words = GROUP * ROW
    mesh = plsc.VectorSubcoreMesh(core_axis_name="c", subcore_axis_name="s")

    @functools.partial(
        pl.kernel,
        out_type=jax.ShapeDtypeStruct((n_rows * ROW,), jnp.int32),
        mesh=mesh,
        scratch_types=[
            pltpu.VMEM((stag_words,), jnp.float32),
            pltpu.VMEM((stag_words,), jnp.int32),
            pltpu.VMEM((T17,), jnp.int32),
            pltpu.VMEM((T16,), jnp.int32),
            pltpu.VMEM((1024 * GROUP,), jnp.int32),
            pltpu.VMEM((512 * GROUP,), jnp.int32),
            pltpu.SemaphoreType.DMA,
            pltpu.SemaphoreType.DMA,
        ],
        compiler_params=pltpu.CompilerParams(needs_layout_passes=False),
    )
    def k(x_hbm, out_hbm, stag_in, stag_out, t17, t16a, hista, histb,
          sem_in, sem_out):
        c = lax.axis_index("c")
        s = lax.axis_index("s")
        wid = s * 2 + c
        iota = lax.iota(jnp.int32, 16)
        zeros = jnp.zeros((16,), jnp.int32)
        ones = jnp.ones((16,), jnp.int32)

        def transform(bits):
            sgn = lax.shift_right_arithmetic(bits, 31)
            m = bits ^ (sgn | INT_MIN)
            return ~m

        def zero_loop(ref, nvregs):
            def body(b, _):
                for m in range(4):
                    ref[pl.ds((b * 4 + m) * 16, 16)] = zeros
                return 0

            lax.fori_loop(0, nvregs // 4, body, 0)

        def scan(ref, off, nbins):
            def body(blk, run):
                for m in range(4):
                    b = off + (blk * 4 + m) * 16
                    v = ref[pl.ds(b, 16)]
                    ref[pl.ds(b, 16)] = run
                    run = run + v
                return run

            lax.fori_loop(0, nbins // 4, body, zeros)

        zero_loop(hista, 1024)
        zero_loop(histb, 512)

        wbase = wid * rows_per_w * ROW
        pltpu.sync_copy(x_hbm.at[pl.ds(wbase, stag_words)], stag_in)

        def do_group(g, _):
            base = wbase + g * stag_words
            nbase = base + stag_words

            def tin_row(r, _):
                def tin_body(j, _):
                    src = r * ROW + j * 16 + iota
                    src = jnp.minimum(src, stag_words - 1)
                    bits = plsc.bitcast(
                        plsc.load_gather(stag_in, [src]), jnp.int32)
                    key = transform(bits)
                    key = jnp.where((j * 16 + iota) < ROW, key, jnp.int32(-1))
                    dst = (j * 16 + iota) * 17 + r
                    plsc.store_scatter(t17, [dst], key)
                    return 0

                lax.fori_loop(0, 63, tin_body, 0)
                return 0

            lax.fori_loop(0, 16, tin_row, 0)

            @pl.when(g + 1 < groups)
            def _():
                pltpu.make_async_copy(
                    x_hbm.at[pl.ds(nbase, stag_words)], stag_in,
                    sem_in).start()

            def ha_body(j, _):
                key = plsc.load_gather(t17, [j * 17 + iota])
                hidx = ((key & LOW10) << 4) + iota
                plsc.addupdate_scatter(hista, [hidx], ones)
                return 0

            lax.fori_loop(0, PPAD, ha_body, 0)
            scan(hista, 0, 1024)

            def permute(src, src_pitch, dst, dst_pitch, cur, hoff, p, kind,
                        blk_sz=4):
                shift, mask = DIGITS[p]
                hvec = iota + hoff

                def load_block(j0):
                    return tuple(
                        plsc.load_gather(src, [(j0 + m) * src_pitch + iota])
                        for m in range(blk_sz))

                def proc_block(j0, keys):
                    ds_ = [lax.shift_right_logical(k_, shift) & mask
                           for k_ in keys]
                    hidxs = [(d << 4) + hvec for d in ds_]
                    gs = [plsc.load_gather(cur, [hidxs[m]])
                          for m in range(blk_sz)]
                    for m in range(blk_sz):
                        pos = gs[m]
                        for mm in range(m):
                            pos = pos + jnp.where(ds_[m] == ds_[mm],
                                                  ones, zeros)
                        dv = pos * dst_pitch + iota
                        if kind == 0:
                            plsc.store_scatter(
                                dst, [dv], (keys[m] & NLOW10) | (j0 + m))
                        elif kind == 1:
                            plsc.store_scatter(dst, [dv], keys[m])
                        else:
                            plsc.store_scatter(dst, [dv], keys[m] & LOW10)
                        plsc.addupdate_scatter(cur, [hidxs[m]], ones)

                def body(blk, carry):
                    nxt = load_block((blk + 1) * blk_sz)
                    proc_block(blk * blk_sz, carry)
                    return nxt

                nblk = PPAD // blk_sz
                last = lax.fori_loop(0, nblk - 1, body, load_block(0))
                proc_block((nblk - 1) * blk_sz, last)

            permute(t17, 17, t16a, 16, hista, 0, 0, 0)

            def hb_body(j, _):
                key = plsc.load_gather(t16a, [j * 16 + iota])
                for p in (1, 2, 3):
                    shift, mask = DIGITS[p]
                    d = lax.shift_right_logical(key, shift) & mask
                    hidx = (d << 4) + iota + HB_OFF[p]
                    plsc.addupdate_scatter(histb, [hidx], ones)
                return 0

            lax.fori_loop(0, PPAD, hb_body, 0)

            scan(histb, 0, 256)
            permute(t16a, 16, t17, 16, histb, HB_OFF[1], 1, 1)
            scan(histb, 4096, 128)
            permute(t17, 16, t16a, 16, histb, HB_OFF[2], 2, 1)
            scan(histb, 6144, 128)
            permute(t16a, 16, t17, 17, histb, HB_OFF[3], 3, 2)

            @pl.when(g > 0)
            def _():
                pltpu.make_async_copy(
                    stag_out, out_hbm.at[pl.ds(base - stag_words, stag_words)],
                    sem_out).wait()

            def tout_row(r, _):
                def tout_body(j, _):
                    src = (j * 16 + iota) * 17 + r
                    v = plsc.load_gather(t17, [src])
                    dst = r * ROW + j * 16 + iota
                    dst = jnp.minimum(dst, stag_words - 1)
                    msk = (j * 16 + iota) < ROW
                    plsc.store_scatter(stag_out, [dst], v, mask=msk)
                    hista[pl.ds((r * 63 + j) * 16, 16)] = zeros
                    return 0

                lax.fori_loop(0, 63, tout_body, 0)
                return 0

            lax.fori_loop(0, 16, tout_row, 0)
            for b in range(1008, 1024):
                hista[pl.ds(b * 16, 16)] = zeros
            zero_loop(histb, 512)

            pltpu.make_async_copy(
                stag_out, out_hbm.at[pl.ds(base, stag_words)],
                sem_out).start()

            @pl.when(g + 1 < groups)
            def _():
                pltpu.make_async_copy(
                    x_hbm.at[pl.ds(nbase, stag_words)], stag_in,
                    sem_in).wait()

            return 0

        lax.fori_loop(0, groups, do_group, 0)
        pltpu.make_async_copy(
            stag_out,
            out_hbm.at[pl.ds(wbase + (groups - 1) * stag_words, stag_words)],
            sem_out).wait()

    return k


@jax.jit
def kernel(inputs):
    n_rows = inputs.size // ROW
    out = _make(n_rows)(inputs)
    return out.reshape(n_rows, ROW)
